# Initial kernel scaffold; baseline (speedup 1.0000x reference)
import jax
import jax.numpy as jnp
from jax.experimental import pallas as pl


def kernel(features, edge_index, etypes, W_lin, b_lin, W_ih, W_hh, b_ih, b_hh, W_i, b_i, W_j, b_j):
    raise NotImplementedError("write your pallas kernel here")


# speedup vs baseline: 9.8459x; 9.8459x over previous
import functools

import jax
import jax.numpy as jnp
from jax import lax
from jax.experimental import pallas as pl
from jax.experimental.pallas import tpu as pltpu
from jax.experimental.pallas import tpu_sc as plsc

N = 10000
E = 320000
D = 128
T = 4
STEPS = 3

NC = 2
NS = 16
DH = D // NC
EPT = E // NS
KCH = 80
NCH = EPT // KCH
NP = 10240
RPT = NP // NS

BN = 400
GRID_N = N // BN


def _wh_body(h_ref, w_ref, b_ref, out_ref):
    h = h_ref[...]
    w = w_ref[0]
    res = lax.dot_general(h, w, (((1,), (1,)), ((), ())),
                          preferred_element_type=jnp.float32) + b_ref[0]
    out_ref[0, 0] = res[:, :DH]
    out_ref[1, 0] = res[:, DH:]


def _compute_wh(h, W_lin, b_lin3):
    return pl.pallas_call(
        _wh_body,
        grid=(T, GRID_N),
        in_specs=[
            pl.BlockSpec((BN, D), lambda t, i: (i, 0)),
            pl.BlockSpec((1, D, D), lambda t, i: (t, 0, 0)),
            pl.BlockSpec((1, 1, D), lambda t, i: (t, 0, 0)),
        ],
        out_specs=pl.BlockSpec((2, 1, BN, DH), lambda t, i: (0, t, i, 0)),
        out_shape=jax.ShapeDtypeStruct((2, T, N, DH), jnp.float32),
    )(h, W_lin, b_lin3)


@functools.partial(
    pl.kernel,
    mesh=plsc.VectorSubcoreMesh(core_axis_name="c", subcore_axis_name="s"),
    compiler_params=pltpu.CompilerParams(use_tc_tiling_on_sc=False),
    out_type=jax.ShapeDtypeStruct((NC, NP, DH), jnp.float32),
    scratch_types=[
        pltpu.VMEM((KCH,), jnp.int32),
        pltpu.VMEM((KCH,), jnp.int32),
        pltpu.VMEM((KCH, DH), jnp.float32),
        pltpu.VMEM((RPT, DH), jnp.float32),
        pltpu.VMEM_SHARED((NP, DH), jnp.float32),
        pltpu.SemaphoreType.DMA,
    ],
)
def _seg_sum(table_hbm, gidx_hbm, didx_hbm, out_hbm,
             gi_v, di_v, rows_v, zer_v, acc_sh, sem):
    cid = lax.axis_index("c")
    sid = lax.axis_index("s")

    def zrow(r, carry):
        for cc in range(DH // 16):
            zer_v[r, pl.ds(cc * 16, 16)] = jnp.zeros((16,), jnp.float32)
        return carry
    lax.fori_loop(0, RPT, zrow, 0)
    pltpu.sync_copy(zer_v, acc_sh.at[pl.ds(sid * RPT, RPT)])
    plsc.subcore_barrier()

    base0 = sid * EPT

    def chunk(c, carry):
        base = pl.multiple_of(base0 + c * KCH, 8)
        pltpu.sync_copy(gidx_hbm.at[pl.ds(base, KCH)], gi_v)
        pltpu.sync_copy(didx_hbm.at[pl.ds(base, KCH)], di_v)
        pltpu.async_copy(table_hbm.at[cid].at[gi_v], rows_v, sem).wait()
        pltpu.sync_copy(rows_v, acc_sh.at[di_v], add=True)
        return carry
    lax.fori_loop(0, NCH, chunk, 0)

    plsc.subcore_barrier()
    pltpu.sync_copy(acc_sh.at[pl.ds(sid * RPT, RPT)],
                    out_hbm.at[cid, pl.ds(sid * RPT, RPT)])


def _gru_body(p0_ref, p1_ref, h_ref, wih_ref, whh_ref, bih_ref, bhh_ref,
              out_ref):
    a = jnp.concatenate((p0_ref[...], p1_ref[...]), axis=1)
    h = h_ref[...]
    gi = lax.dot_general(a, wih_ref[...], (((1,), (1,)), ((), ())),
                         preferred_element_type=jnp.float32) + bih_ref[...]
    gh = lax.dot_general(h, whh_ref[...], (((1,), (1,)), ((), ())),
                         preferred_element_type=jnp.float32) + bhh_ref[...]
    r = jax.nn.sigmoid(gi[:, :D] + gh[:, :D])
    z = jax.nn.sigmoid(gi[:, D:2 * D] + gh[:, D:2 * D])
    n = jnp.tanh(gi[:, 2 * D:] + r * gh[:, 2 * D:])
    out_ref[...] = (1.0 - z) * n + z * h


def _gru(p0, p1, h, W_ih, W_hh, b_ih, b_hh):
    return pl.pallas_call(
        _gru_body,
        grid=(GRID_N,),
        in_specs=[
            pl.BlockSpec((BN, DH), lambda i: (i, 0)),
            pl.BlockSpec((BN, DH), lambda i: (i, 0)),
            pl.BlockSpec((BN, D), lambda i: (i, 0)),
            pl.BlockSpec((3 * D, D), lambda i: (0, 0)),
            pl.BlockSpec((3 * D, D), lambda i: (0, 0)),
            pl.BlockSpec((1, 3 * D), lambda i: (0, 0)),
            pl.BlockSpec((1, 3 * D), lambda i: (0, 0)),
        ],
        out_specs=pl.BlockSpec((BN, D), lambda i: (i, 0)),
        out_shape=jax.ShapeDtypeStruct((N, D), jnp.float32),
    )(p0, p1, h, W_ih, W_hh, b_ih, b_hh)


def _out_body(h_ref, f_ref, wi_ref, bi_ref, wj_ref, bj_ref, out_ref):
    h = h_ref[...]
    f = f_ref[...]
    wi = wi_ref[...]
    g = (lax.dot_general(h, wi[:, :D], (((1,), (1,)), ((), ())),
                         preferred_element_type=jnp.float32)
         + lax.dot_general(f, wi[:, D:], (((1,), (1,)), ((), ())),
                           preferred_element_type=jnp.float32)
         + bi_ref[...])
    gate = jax.nn.sigmoid(g)
    proj = lax.dot_general(h, wj_ref[...], (((1,), (1,)), ((), ())),
                           preferred_element_type=jnp.float32) + bj_ref[...]
    out_ref[...] = gate * proj


def _final(h, features, W_i, b_i, W_j, b_j):
    return pl.pallas_call(
        _out_body,
        grid=(GRID_N,),
        in_specs=[
            pl.BlockSpec((BN, D), lambda i: (i, 0)),
            pl.BlockSpec((BN, D), lambda i: (i, 0)),
            pl.BlockSpec((D, 2 * D), lambda i: (0, 0)),
            pl.BlockSpec((1, D), lambda i: (0, 0)),
            pl.BlockSpec((D, D), lambda i: (0, 0)),
            pl.BlockSpec((1, D), lambda i: (0, 0)),
        ],
        out_specs=pl.BlockSpec((BN, D), lambda i: (i, 0)),
        out_shape=jax.ShapeDtypeStruct((N, D), jnp.float32),
    )(h, features, W_i, b_i, W_j, b_j)


def kernel(features, edge_index, etypes, W_lin, b_lin, W_ih, W_hh, b_ih,
           b_hh, W_i, b_i, W_j, b_j):
    src = edge_index[0].astype(jnp.int32)
    dst = edge_index[1].astype(jnp.int32)
    et = etypes.astype(jnp.int32)
    gidx = et * N + src

    blin3 = b_lin.reshape(T, 1, D)
    bih2 = b_ih.reshape(1, 3 * D)
    bhh2 = b_hh.reshape(1, 3 * D)
    bi2 = b_i.reshape(1, D)
    bj2 = b_j.reshape(1, D)

    h = features
    for _ in range(STEPS):
        wh = _compute_wh(h, W_lin, blin3)
        parts = _seg_sum(wh.reshape(NC, T * N, DH), gidx, dst)
        h = _gru(parts[0, :N], parts[1, :N], h, W_ih, W_hh, bih2, bhh2)
    return _final(h, features, W_i, bi2, W_j, bj2)


# speedup vs baseline: 25.5052x; 2.5904x over previous
import functools

import jax
import jax.numpy as jnp
from jax import lax
from jax.experimental import pallas as pl
from jax.experimental.pallas import tpu as pltpu
from jax.experimental.pallas import tpu_sc as plsc

N = 10000
E = 320000
D = 128
T = 4
STEPS = 3

NC = 2
NS = 16
DH = D // NC
EPT = E // NS
KCH = 80
NCH = EPT // KCH
NP = 10240
RPT = NP // NS

BN = 400
GRID_N = N // BN


def _wh_body(h_ref, w_ref, b_ref, out_ref):
    h = h_ref[...]
    w = w_ref[0]
    res = lax.dot_general(h, w, (((1,), (1,)), ((), ())),
                          preferred_element_type=jnp.float32) + b_ref[0]
    out_ref[0, 0] = res[:, :DH]
    out_ref[1, 0] = res[:, DH:]


def _compute_wh(h, W_lin, b_lin3):
    return pl.pallas_call(
        _wh_body,
        grid=(T, GRID_N),
        in_specs=[
            pl.BlockSpec((BN, D), lambda t, i: (i, 0)),
            pl.BlockSpec((1, D, D), lambda t, i: (t, 0, 0)),
            pl.BlockSpec((1, 1, D), lambda t, i: (t, 0, 0)),
        ],
        out_specs=pl.BlockSpec((2, 1, BN, DH), lambda t, i: (0, t, i, 0)),
        out_shape=jax.ShapeDtypeStruct((2, T, N, DH), jnp.float32),
    )(h, W_lin, b_lin3)


NB = 5
ZR = 128


@functools.partial(
    pl.kernel,
    mesh=plsc.VectorSubcoreMesh(core_axis_name="c", subcore_axis_name="s"),
    compiler_params=pltpu.CompilerParams(use_tc_tiling_on_sc=False),
    out_type=jax.ShapeDtypeStruct((NC, NP, DH), jnp.float32),
    scratch_types=[
        pltpu.VMEM((NCH, KCH), jnp.int32),
        pltpu.VMEM((NCH, KCH), jnp.int32),
        pltpu.VMEM((NB, KCH, DH), jnp.float32),
        pltpu.VMEM((ZR, DH), jnp.float32),
        pltpu.VMEM_SHARED((NP, DH), jnp.float32),
    ] + [pltpu.SemaphoreType.DMA] * NB,
)
def _seg_sum(gidx_hbm, didx_hbm, table_hbm, out_hbm,
             gi_v, di_v, rows_v, zer_v, acc_sh, *gsem):
    cid = lax.axis_index("c")
    sid = lax.axis_index("s")

    pltpu.sync_copy(gidx_hbm.at[sid], gi_v)
    pltpu.sync_copy(didx_hbm.at[sid], di_v)

    def zrow(r, carry):
        for cc in range(DH // 16):
            zer_v[r, pl.ds(cc * 16, 16)] = jnp.zeros((16,), jnp.float32)
        return carry
    lax.fori_loop(0, ZR, zrow, 0)
    for z in range(RPT // ZR):
        pltpu.sync_copy(zer_v, acc_sh.at[pl.ds(sid * RPT + z * ZR, ZR)])
    plsc.subcore_barrier()

    table = table_hbm.at[cid]

    def start_gather(c, b):
        pltpu.async_copy(table.at[gi_v.at[c]], rows_v.at[b], gsem[b])

    def wait_gather(c, b):
        pltpu.make_async_copy(table.at[gi_v.at[c]], rows_v.at[b],
                              gsem[b]).wait()

    for b in range(NB - 1):
        start_gather(b, b)

    def body(i, carry):
        c0 = i * NB
        for b in range(NB):
            c = c0 + b
            nxt = c + NB - 1

            @pl.when(nxt < NCH)
            def _():
                start_gather(nxt, (b + NB - 1) % NB)
            wait_gather(c, b)
            pltpu.sync_copy(rows_v.at[b], acc_sh.at[di_v.at[c]], add=True)
        return carry
    lax.fori_loop(0, NCH // NB, body, 0)

    plsc.subcore_barrier()
    pltpu.sync_copy(acc_sh.at[pl.ds(sid * RPT, RPT)],
                    out_hbm.at[cid, pl.ds(sid * RPT, RPT)])


def _gru_body(p0_ref, p1_ref, h_ref, wih_ref, whh_ref, bih_ref, bhh_ref,
              out_ref):
    a = jnp.concatenate((p0_ref[...], p1_ref[...]), axis=1)
    h = h_ref[...]
    gi = lax.dot_general(a, wih_ref[...], (((1,), (1,)), ((), ())),
                         preferred_element_type=jnp.float32) + bih_ref[...]
    gh = lax.dot_general(h, whh_ref[...], (((1,), (1,)), ((), ())),
                         preferred_element_type=jnp.float32) + bhh_ref[...]
    r = jax.nn.sigmoid(gi[:, :D] + gh[:, :D])
    z = jax.nn.sigmoid(gi[:, D:2 * D] + gh[:, D:2 * D])
    n = jnp.tanh(gi[:, 2 * D:] + r * gh[:, 2 * D:])
    out_ref[...] = (1.0 - z) * n + z * h


def _gru(p0, p1, h, W_ih, W_hh, b_ih, b_hh):
    return pl.pallas_call(
        _gru_body,
        grid=(GRID_N,),
        in_specs=[
            pl.BlockSpec((BN, DH), lambda i: (i, 0)),
            pl.BlockSpec((BN, DH), lambda i: (i, 0)),
            pl.BlockSpec((BN, D), lambda i: (i, 0)),
            pl.BlockSpec((3 * D, D), lambda i: (0, 0)),
            pl.BlockSpec((3 * D, D), lambda i: (0, 0)),
            pl.BlockSpec((1, 3 * D), lambda i: (0, 0)),
            pl.BlockSpec((1, 3 * D), lambda i: (0, 0)),
        ],
        out_specs=pl.BlockSpec((BN, D), lambda i: (i, 0)),
        out_shape=jax.ShapeDtypeStruct((N, D), jnp.float32),
    )(p0, p1, h, W_ih, W_hh, b_ih, b_hh)


def _out_body(h_ref, f_ref, wi_ref, bi_ref, wj_ref, bj_ref, out_ref):
    h = h_ref[...]
    f = f_ref[...]
    wi = wi_ref[...]
    g = (lax.dot_general(h, wi[:, :D], (((1,), (1,)), ((), ())),
                         preferred_element_type=jnp.float32)
         + lax.dot_general(f, wi[:, D:], (((1,), (1,)), ((), ())),
                           preferred_element_type=jnp.float32)
         + bi_ref[...])
    gate = jax.nn.sigmoid(g)
    proj = lax.dot_general(h, wj_ref[...], (((1,), (1,)), ((), ())),
                           preferred_element_type=jnp.float32) + bj_ref[...]
    out_ref[...] = gate * proj


def _final(h, features, W_i, b_i, W_j, b_j):
    return pl.pallas_call(
        _out_body,
        grid=(GRID_N,),
        in_specs=[
            pl.BlockSpec((BN, D), lambda i: (i, 0)),
            pl.BlockSpec((BN, D), lambda i: (i, 0)),
            pl.BlockSpec((D, 2 * D), lambda i: (0, 0)),
            pl.BlockSpec((1, D), lambda i: (0, 0)),
            pl.BlockSpec((D, D), lambda i: (0, 0)),
            pl.BlockSpec((1, D), lambda i: (0, 0)),
        ],
        out_specs=pl.BlockSpec((BN, D), lambda i: (i, 0)),
        out_shape=jax.ShapeDtypeStruct((N, D), jnp.float32),
    )(h, features, W_i, b_i, W_j, b_j)


def kernel(features, edge_index, etypes, W_lin, b_lin, W_ih, W_hh, b_ih,
           b_hh, W_i, b_i, W_j, b_j):
    src = edge_index[0].astype(jnp.int32)
    dst = edge_index[1].astype(jnp.int32)
    et = etypes.astype(jnp.int32)
    gidx = (et * N + src).reshape(NS, NCH, KCH)
    dst3 = dst.reshape(NS, NCH, KCH)

    blin3 = b_lin.reshape(T, 1, D)
    bih2 = b_ih.reshape(1, 3 * D)
    bhh2 = b_hh.reshape(1, 3 * D)
    bi2 = b_i.reshape(1, D)
    bj2 = b_j.reshape(1, D)

    h = features
    for _ in range(STEPS):
        wh = _compute_wh(h, W_lin, blin3)
        parts = _seg_sum(gidx, dst3, wh.reshape(NC, T * N, DH))
        h = _gru(parts[0, :N], parts[1, :N], h, W_ih, W_hh, bih2, bhh2)
    return _final(h, features, W_i, bi2, W_j, bj2)


# speedup vs baseline: 34.9243x; 1.3693x over previous
import functools

import jax
import jax.numpy as jnp
from jax import lax
from jax.experimental import pallas as pl
from jax.experimental.pallas import tpu as pltpu
from jax.experimental.pallas import tpu_sc as plsc

N = 10000
E = 320000
D = 128
T = 4
STEPS = 3

NC = 2
NS = 16
DH = D // NC
EPT = E // NS
KCH = 80
NCH = EPT // KCH
NB = 5
NP = 12800
NZ = 10240
RPT = NZ // NS
ZR = 160

BN = 400
GRID_N = N // BN
NPB = NP // BN


@functools.partial(
    pl.kernel,
    mesh=plsc.VectorSubcoreMesh(core_axis_name="c", subcore_axis_name="s"),
    compiler_params=pltpu.CompilerParams(use_tc_tiling_on_sc=False),
    out_type=jax.ShapeDtypeStruct((NC, NP, DH), jnp.float32),
    scratch_types=[
        pltpu.VMEM((NCH, KCH), jnp.int32),
        pltpu.VMEM((NCH, KCH), jnp.int32),
        pltpu.VMEM((NB, KCH, DH), jnp.float32),
        pltpu.VMEM((ZR, DH), jnp.float32),
        pltpu.VMEM_SHARED((NP, DH), jnp.float32),
    ] + [pltpu.SemaphoreType.DMA] * NB,
)
def _seg_sum(gidx_hbm, didx_hbm, table_hbm, out_hbm,
             gi_v, di_v, rows_v, zer_v, acc_sh, *gsem):
    cid = lax.axis_index("c")
    sid = lax.axis_index("s")

    pltpu.sync_copy(gidx_hbm.at[sid], gi_v)
    pltpu.sync_copy(didx_hbm.at[sid], di_v)

    def zrow(r, carry):
        for cc in range(DH // 16):
            zer_v[r, pl.ds(cc * 16, 16)] = jnp.zeros((16,), jnp.float32)
        return carry
    lax.fori_loop(0, ZR, zrow, 0)
    for z in range(RPT // ZR):
        pltpu.sync_copy(zer_v, acc_sh.at[pl.ds(sid * RPT + z * ZR, ZR)])
    plsc.subcore_barrier()

    table = table_hbm.at[cid]

    def start_gather(c, b):
        pltpu.async_copy(table.at[gi_v.at[c]], rows_v.at[b], gsem[b])

    def wait_gather(c, b):
        pltpu.make_async_copy(table.at[gi_v.at[c]], rows_v.at[b],
                              gsem[b]).wait()

    for b in range(NB - 1):
        start_gather(b, b)

    def body(i, carry):
        c0 = i * NB
        for b in range(NB):
            c = c0 + b
            nxt = c + NB - 1

            @pl.when(nxt < NCH)
            def _():
                start_gather(nxt, (b + NB - 1) % NB)
            wait_gather(c, b)
            pltpu.sync_copy(rows_v.at[b], acc_sh.at[di_v.at[c]], add=True)
        return carry
    lax.fori_loop(0, NCH // NB, body, 0)

    plsc.subcore_barrier()
    pltpu.sync_copy(acc_sh.at[pl.ds(sid * RPT, RPT)],
                    out_hbm.at[cid, pl.ds(sid * RPT, RPT)])


def _etype_linear(h, wlA_ref, wlB_ref, blA_ref, blB_ref, whA_ref, whB_ref):
    for t in range(T):
        whA_ref[t] = lax.dot_general(
            h, wlA_ref[t], (((1,), (1,)), ((), ())),
            preferred_element_type=jnp.float32) + blA_ref[t]
        whB_ref[t] = lax.dot_general(
            h, wlB_ref[t], (((1,), (1,)), ((), ())),
            preferred_element_type=jnp.float32) + blB_ref[t]


def _gru_math(p0, p1, h, wih, whh, bih, bhh):
    a = jnp.concatenate((p0, p1), axis=1)
    gi = lax.dot_general(a, wih, (((1,), (1,)), ((), ())),
                         preferred_element_type=jnp.float32) + bih
    gh = lax.dot_general(h, whh, (((1,), (1,)), ((), ())),
                         preferred_element_type=jnp.float32) + bhh
    r = jax.nn.sigmoid(gi[:, :D] + gh[:, :D])
    z = jax.nn.sigmoid(gi[:, D:2 * D] + gh[:, D:2 * D])
    n = jnp.tanh(gi[:, 2 * D:] + r * gh[:, 2 * D:])
    return (1.0 - z) * n + z * h


_WSPEC = lambda *shape: pl.BlockSpec(shape, lambda i: (0,) * len(shape))


def _wh0_body(h_ref, wlA_ref, wlB_ref, blA_ref, blB_ref, out_ref):
    _etype_linear(h_ref[...], wlA_ref, wlB_ref, blA_ref, blB_ref,
                  out_ref.at[0], out_ref.at[1])


def _wh0(h, wlA, wlB, blA, blB):
    return pl.pallas_call(
        _wh0_body,
        grid=(GRID_N,),
        in_specs=[
            pl.BlockSpec((BN, D), lambda i: (i, 0)),
            _WSPEC(T, DH, D),
            _WSPEC(T, DH, D),
            _WSPEC(T, 1, DH),
            _WSPEC(T, 1, DH),
        ],
        out_specs=pl.BlockSpec((2, T, BN, DH), lambda i: (0, 0, i, 0)),
        out_shape=jax.ShapeDtypeStruct((2, T, N, DH), jnp.float32),
    )(h, wlA, wlB, blA, blB)


def _gru_wh_body(p0_ref, p1_ref, h_ref, wih_ref, whh_ref, bih_ref, bhh_ref,
                 wlA_ref, wlB_ref, blA_ref, blB_ref, hn_ref, wh_ref):
    hn = _gru_math(p0_ref[...], p1_ref[...], h_ref[...], wih_ref[...],
                   whh_ref[...], bih_ref[...], bhh_ref[...])
    hn_ref[...] = hn
    _etype_linear(hn, wlA_ref, wlB_ref, blA_ref, blB_ref,
                  wh_ref.at[0], wh_ref.at[1])


def _gru_wh(parts2, h, W_ih, W_hh, bih2, bhh2, wlA, wlB, blA, blB):
    return pl.pallas_call(
        _gru_wh_body,
        grid=(GRID_N,),
        in_specs=[
            pl.BlockSpec((BN, DH), lambda i: (i, 0)),
            pl.BlockSpec((BN, DH), lambda i: (NPB + i, 0)),
            pl.BlockSpec((BN, D), lambda i: (i, 0)),
            _WSPEC(3 * D, D),
            _WSPEC(3 * D, D),
            _WSPEC(1, 3 * D),
            _WSPEC(1, 3 * D),
            _WSPEC(T, DH, D),
            _WSPEC(T, DH, D),
            _WSPEC(T, 1, DH),
            _WSPEC(T, 1, DH),
        ],
        out_specs=[
            pl.BlockSpec((BN, D), lambda i: (i, 0)),
            pl.BlockSpec((2, T, BN, DH), lambda i: (0, 0, i, 0)),
        ],
        out_shape=[
            jax.ShapeDtypeStruct((N, D), jnp.float32),
            jax.ShapeDtypeStruct((2, T, N, DH), jnp.float32),
        ],
    )(parts2, parts2, h, W_ih, W_hh, bih2, bhh2, wlA, wlB, blA, blB)


def _gru_out_body(p0_ref, p1_ref, h_ref, f_ref, wih_ref, whh_ref, bih_ref,
                  bhh_ref, wiA_ref, wiB_ref, bi_ref, wj_ref, bj_ref, out_ref):
    hn = _gru_math(p0_ref[...], p1_ref[...], h_ref[...], wih_ref[...],
                   whh_ref[...], bih_ref[...], bhh_ref[...])
    f = f_ref[...]
    g = (lax.dot_general(hn, wiA_ref[...], (((1,), (1,)), ((), ())),
                         preferred_element_type=jnp.float32)
         + lax.dot_general(f, wiB_ref[...], (((1,), (1,)), ((), ())),
                           preferred_element_type=jnp.float32)
         + bi_ref[...])
    gate = jax.nn.sigmoid(g)
    proj = lax.dot_general(hn, wj_ref[...], (((1,), (1,)), ((), ())),
                           preferred_element_type=jnp.float32) + bj_ref[...]
    out_ref[...] = gate * proj


def _gru_out(parts2, h, features, W_ih, W_hh, bih2, bhh2, wiA, wiB, bi2,
             W_j, bj2):
    return pl.pallas_call(
        _gru_out_body,
        grid=(GRID_N,),
        in_specs=[
            pl.BlockSpec((BN, DH), lambda i: (i, 0)),
            pl.BlockSpec((BN, DH), lambda i: (NPB + i, 0)),
            pl.BlockSpec((BN, D), lambda i: (i, 0)),
            pl.BlockSpec((BN, D), lambda i: (i, 0)),
            _WSPEC(3 * D, D),
            _WSPEC(3 * D, D),
            _WSPEC(1, 3 * D),
            _WSPEC(1, 3 * D),
            _WSPEC(D, D),
            _WSPEC(D, D),
            _WSPEC(1, D),
            _WSPEC(D, D),
            _WSPEC(1, D),
        ],
        out_specs=pl.BlockSpec((BN, D), lambda i: (i, 0)),
        out_shape=jax.ShapeDtypeStruct((N, D), jnp.float32),
    )(parts2, parts2, h, features, W_ih, W_hh, bih2, bhh2, wiA, wiB, bi2,
      W_j, bj2)


def kernel(features, edge_index, etypes, W_lin, b_lin, W_ih, W_hh, b_ih,
           b_hh, W_i, b_i, W_j, b_j):
    src = edge_index[0].astype(jnp.int32)
    dst = edge_index[1].astype(jnp.int32)
    et = etypes.astype(jnp.int32)
    gidx = (et * N + src).reshape(NS, NCH, KCH)
    dst3 = dst.reshape(NS, NCH, KCH)

    wlA = W_lin[:, :DH, :]
    wlB = W_lin[:, DH:, :]
    blA = b_lin[:, :DH].reshape(T, 1, DH)
    blB = b_lin[:, DH:].reshape(T, 1, DH)
    bih2 = b_ih.reshape(1, 3 * D)
    bhh2 = b_hh.reshape(1, 3 * D)
    wiA = W_i[:, :D]
    wiB = W_i[:, D:]
    bi2 = b_i.reshape(1, D)
    bj2 = b_j.reshape(1, D)

    h = features
    wh = _wh0(h, wlA, wlB, blA, blB)
    for s in range(STEPS):
        parts = _seg_sum(gidx, dst3, wh.reshape(NC, T * N, DH))
        parts2 = parts.reshape(NC * NP, DH)
        if s < STEPS - 1:
            h, wh = _gru_wh(parts2, h, W_ih, W_hh, bih2, bhh2,
                            wlA, wlB, blA, blB)
        else:
            return _gru_out(parts2, h, features, W_ih, W_hh, bih2, bhh2,
                            wiA, wiB, bi2, W_j, bj2)


# speedup vs baseline: 34.9902x; 1.0019x over previous
import functools

import jax
import jax.numpy as jnp
from jax import lax
from jax.experimental import pallas as pl
from jax.experimental.pallas import tpu as pltpu
from jax.experimental.pallas import tpu_sc as plsc

N = 10000
E = 320000
D = 128
T = 4
STEPS = 3

NC = 2
NS = 16
DH = D // NC
EPT = E // NS
KCH = 80
NCH = EPT // KCH
NB = 5
NP = 12800
NZ = 10240
RPT = NZ // NS
ZR = 160

BN = 400
GRID_N = N // BN
NPB = NP // BN


@functools.partial(
    pl.kernel,
    mesh=plsc.VectorSubcoreMesh(core_axis_name="c", subcore_axis_name="s"),
    compiler_params=pltpu.CompilerParams(use_tc_tiling_on_sc=False),
    out_type=jax.ShapeDtypeStruct((NC, NP, DH), jnp.float32),
    scratch_types=[
        pltpu.VMEM((NCH, KCH), jnp.int32),
        pltpu.VMEM((NCH, KCH), jnp.int32),
        pltpu.VMEM((NB, KCH, DH), jnp.float32),
        pltpu.VMEM((ZR, DH), jnp.float32),
        pltpu.VMEM_SHARED((NP, DH), jnp.float32),
    ] + [pltpu.SemaphoreType.DMA] * NB,
)
def _seg_sum(gidx_hbm, didx_hbm, table_hbm, out_hbm,
             gi_v, di_v, rows_v, zer_v, acc_sh, *gsem):
    cid = lax.axis_index("c")
    sid = lax.axis_index("s")

    pltpu.sync_copy(gidx_hbm.at[sid], gi_v)
    pltpu.sync_copy(didx_hbm.at[sid], di_v)

    def zrow(r, carry):
        for cc in range(DH // 16):
            zer_v[r, pl.ds(cc * 16, 16)] = jnp.zeros((16,), jnp.float32)
        return carry
    lax.fori_loop(0, ZR, zrow, 0)
    for z in range(RPT // ZR):
        pltpu.sync_copy(zer_v, acc_sh.at[pl.ds(sid * RPT + z * ZR, ZR)])
    plsc.subcore_barrier()

    table = table_hbm.at[cid]

    def start_gather(c, b):
        pltpu.async_copy(table.at[gi_v.at[c]], rows_v.at[b], gsem[b])

    def wait_gather(c, b):
        pltpu.make_async_copy(table.at[gi_v.at[c]], rows_v.at[b],
                              gsem[b]).wait()

    for b in range(NB - 1):
        start_gather(b, b)

    def body(i, carry):
        c0 = i * NB
        for b in range(NB):
            c = c0 + b
            nxt = c + NB - 1

            @pl.when(nxt < NCH)
            def _():
                start_gather(nxt, (b + NB - 1) % NB)
            wait_gather(c, b)
            pltpu.sync_copy(rows_v.at[b], acc_sh.at[di_v.at[c]], add=True)
        return carry
    lax.fori_loop(0, NCH // NB, body, 0)

    plsc.subcore_barrier()
    pltpu.sync_copy(acc_sh.at[pl.ds(sid * RPT, RPT)],
                    out_hbm.at[cid, pl.ds(sid * RPT, RPT)])


def _etype_linear(h, wl_ref, bl_ref, whA_ref, whB_ref):
    for t in range(T):
        res = lax.dot_general(
            h, wl_ref[t], (((1,), (1,)), ((), ())),
            preferred_element_type=jnp.float32) + bl_ref[t]
        whA_ref[t] = res[:, :DH]
        whB_ref[t] = res[:, DH:]


def _gru_math(p0, p1, h, wih, whh, bih, bhh):
    a = jnp.concatenate((p0, p1), axis=1)
    gi = lax.dot_general(a, wih, (((1,), (1,)), ((), ())),
                         preferred_element_type=jnp.float32) + bih
    gh = lax.dot_general(h, whh, (((1,), (1,)), ((), ())),
                         preferred_element_type=jnp.float32) + bhh
    r = jax.nn.sigmoid(gi[:, :D] + gh[:, :D])
    z = jax.nn.sigmoid(gi[:, D:2 * D] + gh[:, D:2 * D])
    n = jnp.tanh(gi[:, 2 * D:] + r * gh[:, 2 * D:])
    return (1.0 - z) * n + z * h


_WSPEC = lambda *shape: pl.BlockSpec(shape, lambda i: (0,) * len(shape))


def _wh0_body(h_ref, wl_ref, bl_ref, out_ref):
    _etype_linear(h_ref[...], wl_ref, bl_ref, out_ref.at[0], out_ref.at[1])


def _wh0(h, W_lin, blin3):
    return pl.pallas_call(
        _wh0_body,
        grid=(GRID_N,),
        in_specs=[
            pl.BlockSpec((BN, D), lambda i: (i, 0)),
            _WSPEC(T, D, D),
            _WSPEC(T, 1, D),
        ],
        out_specs=pl.BlockSpec((2, T, BN, DH), lambda i: (0, 0, i, 0)),
        out_shape=jax.ShapeDtypeStruct((2, T, N, DH), jnp.float32),
    )(h, W_lin, blin3)


def _gru_wh_body(p0_ref, p1_ref, h_ref, wih_ref, whh_ref, bih_ref, bhh_ref,
                 wl_ref, bl_ref, hn_ref, wh_ref):
    hn = _gru_math(p0_ref[...], p1_ref[...], h_ref[...], wih_ref[...],
                   whh_ref[...], bih_ref[...], bhh_ref[...])
    hn_ref[...] = hn
    _etype_linear(hn, wl_ref, bl_ref, wh_ref.at[0], wh_ref.at[1])


def _gru_wh(parts2, h, W_ih, W_hh, bih2, bhh2, W_lin, blin3):
    return pl.pallas_call(
        _gru_wh_body,
        grid=(GRID_N,),
        in_specs=[
            pl.BlockSpec((BN, DH), lambda i: (i, 0)),
            pl.BlockSpec((BN, DH), lambda i: (NPB + i, 0)),
            pl.BlockSpec((BN, D), lambda i: (i, 0)),
            _WSPEC(3 * D, D),
            _WSPEC(3 * D, D),
            _WSPEC(1, 3 * D),
            _WSPEC(1, 3 * D),
            _WSPEC(T, D, D),
            _WSPEC(T, 1, D),
        ],
        out_specs=[
            pl.BlockSpec((BN, D), lambda i: (i, 0)),
            pl.BlockSpec((2, T, BN, DH), lambda i: (0, 0, i, 0)),
        ],
        out_shape=[
            jax.ShapeDtypeStruct((N, D), jnp.float32),
            jax.ShapeDtypeStruct((2, T, N, DH), jnp.float32),
        ],
    )(parts2, parts2, h, W_ih, W_hh, bih2, bhh2, W_lin, blin3)


def _gru_out_body(p0_ref, p1_ref, h_ref, f_ref, wih_ref, whh_ref, bih_ref,
                  bhh_ref, wiA_ref, wiB_ref, bi_ref, wj_ref, bj_ref, out_ref):
    hn = _gru_math(p0_ref[...], p1_ref[...], h_ref[...], wih_ref[...],
                   whh_ref[...], bih_ref[...], bhh_ref[...])
    f = f_ref[...]
    g = (lax.dot_general(hn, wiA_ref[...], (((1,), (1,)), ((), ())),
                         preferred_element_type=jnp.float32)
         + lax.dot_general(f, wiB_ref[...], (((1,), (1,)), ((), ())),
                           preferred_element_type=jnp.float32)
         + bi_ref[...])
    gate = jax.nn.sigmoid(g)
    proj = lax.dot_general(hn, wj_ref[...], (((1,), (1,)), ((), ())),
                           preferred_element_type=jnp.float32) + bj_ref[...]
    out_ref[...] = gate * proj


def _gru_out(parts2, h, features, W_ih, W_hh, bih2, bhh2, wiA, wiB, bi2,
             W_j, bj2):
    return pl.pallas_call(
        _gru_out_body,
        grid=(GRID_N,),
        in_specs=[
            pl.BlockSpec((BN, DH), lambda i: (i, 0)),
            pl.BlockSpec((BN, DH), lambda i: (NPB + i, 0)),
            pl.BlockSpec((BN, D), lambda i: (i, 0)),
            pl.BlockSpec((BN, D), lambda i: (i, 0)),
            _WSPEC(3 * D, D),
            _WSPEC(3 * D, D),
            _WSPEC(1, 3 * D),
            _WSPEC(1, 3 * D),
            _WSPEC(D, D),
            _WSPEC(D, D),
            _WSPEC(1, D),
            _WSPEC(D, D),
            _WSPEC(1, D),
        ],
        out_specs=pl.BlockSpec((BN, D), lambda i: (i, 0)),
        out_shape=jax.ShapeDtypeStruct((N, D), jnp.float32),
    )(parts2, parts2, h, features, W_ih, W_hh, bih2, bhh2, wiA, wiB, bi2,
      W_j, bj2)


def kernel(features, edge_index, etypes, W_lin, b_lin, W_ih, W_hh, b_ih,
           b_hh, W_i, b_i, W_j, b_j):
    src = edge_index[0].astype(jnp.int32)
    dst = edge_index[1].astype(jnp.int32)
    et = etypes.astype(jnp.int32)
    gidx = (et * N + src).reshape(NS, NCH, KCH)
    dst3 = dst.reshape(NS, NCH, KCH)

    blin3 = b_lin.reshape(T, 1, D)
    bih2 = b_ih.reshape(1, 3 * D)
    bhh2 = b_hh.reshape(1, 3 * D)
    wiA = W_i[:, :D]
    wiB = W_i[:, D:]
    bi2 = b_i.reshape(1, D)
    bj2 = b_j.reshape(1, D)

    h = features
    wh = _wh0(h, W_lin, blin3)
    for s in range(STEPS):
        parts = _seg_sum(gidx, dst3, wh.reshape(NC, T * N, DH))
        parts2 = parts.reshape(NC * NP, DH)
        if s < STEPS - 1:
            h, wh = _gru_wh(parts2, h, W_ih, W_hh, bih2, bhh2,
                            W_lin, blin3)
        else:
            return _gru_out(parts2, h, features, W_ih, W_hh, bih2, bhh2,
                            wiA, wiB, bi2, W_j, bj2)


# speedup vs baseline: 35.2353x; 1.0070x over previous
import functools

import jax
import jax.numpy as jnp
from jax import lax
from jax.experimental import pallas as pl
from jax.experimental.pallas import tpu as pltpu
from jax.experimental.pallas import tpu_sc as plsc

N = 10000
E = 320000
D = 128
T = 4
STEPS = 3

NC = 2
NS = 16
DH = D // NC
EPT = E // NS
KCH = 80
NCH = EPT // KCH
NB = 5
NP = 12800
NZ = 10240
RPT = NZ // NS
ZR = 160

BN = 400
GRID_N = N // BN
NPB = NP // BN


@functools.partial(
    pl.kernel,
    mesh=plsc.VectorSubcoreMesh(core_axis_name="c", subcore_axis_name="s"),
    compiler_params=pltpu.CompilerParams(use_tc_tiling_on_sc=False),
    out_type=jax.ShapeDtypeStruct((NC, NP, DH), jnp.float32),
    scratch_types=[
        pltpu.VMEM((NCH, KCH), jnp.int32),
        pltpu.VMEM((NCH, KCH), jnp.int32),
        pltpu.VMEM((NB, KCH, DH), jnp.float32),
        pltpu.VMEM((ZR, DH), jnp.float32),
        pltpu.VMEM_SHARED((NP, DH), jnp.float32),
    ] + [pltpu.SemaphoreType.DMA] * NB,
)
def _seg_sum(gidx_hbm, didx_hbm, table_hbm, out_hbm,
             gi_v, di_v, rows_v, zer_v, acc_sh, *gsem):
    cid = lax.axis_index("c")
    sid = lax.axis_index("s")

    pltpu.sync_copy(gidx_hbm.at[sid], gi_v)
    pltpu.sync_copy(didx_hbm.at[sid], di_v)

    table = table_hbm.at[cid]

    def start_gather(c, b):
        pltpu.async_copy(table.at[gi_v.at[c]], rows_v.at[b], gsem[b])

    def wait_gather(c, b):
        pltpu.make_async_copy(table.at[gi_v.at[c]], rows_v.at[b],
                              gsem[b]).wait()

    for b in range(NB - 1):
        start_gather(b, b)

    def zrow(r, carry):
        for cc in range(DH // 16):
            zer_v[r, pl.ds(cc * 16, 16)] = jnp.zeros((16,), jnp.float32)
        return carry
    lax.fori_loop(0, ZR, zrow, 0)
    for z in range(RPT // ZR):
        pltpu.sync_copy(zer_v, acc_sh.at[pl.ds(sid * RPT + z * ZR, ZR)])
    plsc.subcore_barrier()


    def body(i, carry):
        c0 = i * NB
        for b in range(NB):
            c = c0 + b
            nxt = c + NB - 1

            @pl.when(nxt < NCH)
            def _():
                start_gather(nxt, (b + NB - 1) % NB)
            wait_gather(c, b)
            pltpu.sync_copy(rows_v.at[b], acc_sh.at[di_v.at[c]], add=True)
        return carry
    lax.fori_loop(0, NCH // NB, body, 0)

    plsc.subcore_barrier()
    pltpu.sync_copy(acc_sh.at[pl.ds(sid * RPT, RPT)],
                    out_hbm.at[cid, pl.ds(sid * RPT, RPT)])


def _etype_linear(h, wl_ref, bl_ref, whA_ref, whB_ref):
    for t in range(T):
        res = lax.dot_general(
            h, wl_ref[t], (((1,), (1,)), ((), ())),
            preferred_element_type=jnp.float32) + bl_ref[t]
        whA_ref[t] = res[:, :DH]
        whB_ref[t] = res[:, DH:]


def _gru_math(p0, p1, h, wih, whh, bih, bhh):
    a = jnp.concatenate((p0, p1), axis=1)
    gi = lax.dot_general(a, wih, (((1,), (1,)), ((), ())),
                         preferred_element_type=jnp.float32) + bih
    gh = lax.dot_general(h, whh, (((1,), (1,)), ((), ())),
                         preferred_element_type=jnp.float32) + bhh
    r = jax.nn.sigmoid(gi[:, :D] + gh[:, :D])
    z = jax.nn.sigmoid(gi[:, D:2 * D] + gh[:, D:2 * D])
    n = jnp.tanh(gi[:, 2 * D:] + r * gh[:, 2 * D:])
    return (1.0 - z) * n + z * h


_WSPEC = lambda *shape: pl.BlockSpec(shape, lambda i: (0,) * len(shape))


def _wh0_body(h_ref, wl_ref, bl_ref, out_ref):
    _etype_linear(h_ref[...], wl_ref, bl_ref, out_ref.at[0], out_ref.at[1])


def _wh0(h, W_lin, blin3):
    return pl.pallas_call(
        _wh0_body,
        grid=(GRID_N,),
        in_specs=[
            pl.BlockSpec((BN, D), lambda i: (i, 0)),
            _WSPEC(T, D, D),
            _WSPEC(T, 1, D),
        ],
        out_specs=pl.BlockSpec((2, T, BN, DH), lambda i: (0, 0, i, 0)),
        out_shape=jax.ShapeDtypeStruct((2, T, N, DH), jnp.float32),
    )(h, W_lin, blin3)


def _gru_wh_body(p0_ref, p1_ref, h_ref, wih_ref, whh_ref, bih_ref, bhh_ref,
                 wl_ref, bl_ref, hn_ref, wh_ref):
    hn = _gru_math(p0_ref[...], p1_ref[...], h_ref[...], wih_ref[...],
                   whh_ref[...], bih_ref[...], bhh_ref[...])
    hn_ref[...] = hn
    _etype_linear(hn, wl_ref, bl_ref, wh_ref.at[0], wh_ref.at[1])


def _gru_wh(parts2, h, W_ih, W_hh, bih2, bhh2, W_lin, blin3):
    return pl.pallas_call(
        _gru_wh_body,
        grid=(GRID_N,),
        in_specs=[
            pl.BlockSpec((BN, DH), lambda i: (i, 0)),
            pl.BlockSpec((BN, DH), lambda i: (NPB + i, 0)),
            pl.BlockSpec((BN, D), lambda i: (i, 0)),
            _WSPEC(3 * D, D),
            _WSPEC(3 * D, D),
            _WSPEC(1, 3 * D),
            _WSPEC(1, 3 * D),
            _WSPEC(T, D, D),
            _WSPEC(T, 1, D),
        ],
        out_specs=[
            pl.BlockSpec((BN, D), lambda i: (i, 0)),
            pl.BlockSpec((2, T, BN, DH), lambda i: (0, 0, i, 0)),
        ],
        out_shape=[
            jax.ShapeDtypeStruct((N, D), jnp.float32),
            jax.ShapeDtypeStruct((2, T, N, DH), jnp.float32),
        ],
    )(parts2, parts2, h, W_ih, W_hh, bih2, bhh2, W_lin, blin3)


def _gru_out_body(p0_ref, p1_ref, h_ref, f_ref, wih_ref, whh_ref, bih_ref,
                  bhh_ref, wiA_ref, wiB_ref, bi_ref, wj_ref, bj_ref, out_ref):
    hn = _gru_math(p0_ref[...], p1_ref[...], h_ref[...], wih_ref[...],
                   whh_ref[...], bih_ref[...], bhh_ref[...])
    f = f_ref[...]
    g = (lax.dot_general(hn, wiA_ref[...], (((1,), (1,)), ((), ())),
                         preferred_element_type=jnp.float32)
         + lax.dot_general(f, wiB_ref[...], (((1,), (1,)), ((), ())),
                           preferred_element_type=jnp.float32)
         + bi_ref[...])
    gate = jax.nn.sigmoid(g)
    proj = lax.dot_general(hn, wj_ref[...], (((1,), (1,)), ((), ())),
                           preferred_element_type=jnp.float32) + bj_ref[...]
    out_ref[...] = gate * proj


def _gru_out(parts2, h, features, W_ih, W_hh, bih2, bhh2, wiA, wiB, bi2,
             W_j, bj2):
    return pl.pallas_call(
        _gru_out_body,
        grid=(GRID_N,),
        in_specs=[
            pl.BlockSpec((BN, DH), lambda i: (i, 0)),
            pl.BlockSpec((BN, DH), lambda i: (NPB + i, 0)),
            pl.BlockSpec((BN, D), lambda i: (i, 0)),
            pl.BlockSpec((BN, D), lambda i: (i, 0)),
            _WSPEC(3 * D, D),
            _WSPEC(3 * D, D),
            _WSPEC(1, 3 * D),
            _WSPEC(1, 3 * D),
            _WSPEC(D, D),
            _WSPEC(D, D),
            _WSPEC(1, D),
            _WSPEC(D, D),
            _WSPEC(1, D),
        ],
        out_specs=pl.BlockSpec((BN, D), lambda i: (i, 0)),
        out_shape=jax.ShapeDtypeStruct((N, D), jnp.float32),
    )(parts2, parts2, h, features, W_ih, W_hh, bih2, bhh2, wiA, wiB, bi2,
      W_j, bj2)


def kernel(features, edge_index, etypes, W_lin, b_lin, W_ih, W_hh, b_ih,
           b_hh, W_i, b_i, W_j, b_j):
    src = edge_index[0].astype(jnp.int32)
    dst = edge_index[1].astype(jnp.int32)
    et = etypes.astype(jnp.int32)
    gidx = (et * N + src).reshape(NS, NCH, KCH)
    dst3 = dst.reshape(NS, NCH, KCH)

    blin3 = b_lin.reshape(T, 1, D)
    bih2 = b_ih.reshape(1, 3 * D)
    bhh2 = b_hh.reshape(1, 3 * D)
    wiA = W_i[:, :D]
    wiB = W_i[:, D:]
    bi2 = b_i.reshape(1, D)
    bj2 = b_j.reshape(1, D)

    h = features
    wh = _wh0(h, W_lin, blin3)
    for s in range(STEPS):
        parts = _seg_sum(gidx, dst3, wh.reshape(NC, T * N, DH))
        parts2 = parts.reshape(NC * NP, DH)
        if s < STEPS - 1:
            h, wh = _gru_wh(parts2, h, W_ih, W_hh, bih2, bhh2,
                            W_lin, blin3)
        else:
            return _gru_out(parts2, h, features, W_ih, W_hh, bih2, bhh2,
                            wiA, wiB, bi2, W_j, bj2)
